# baseline (speedup 1.0000x reference)
import jax
import jax.numpy as jnp
from jax.experimental import pallas as pl

_TN = 512


def _moe_head_kernel(x_ref, w1_ref, b1_ref, w2_ref, b2_ref,
                     mixed_ref, el_ref, probs_ref):
    x = x_ref[...].astype(jnp.bfloat16)
    e_count = w1_ref.shape[0]
    acc = None
    for e in range(e_count):
        h = jnp.dot(x, w1_ref[e], preferred_element_type=jnp.float32)
        h = jnp.maximum(h + b1_ref[e][None, :], 0.0).astype(jnp.bfloat16)
        lg = jnp.dot(h, w2_ref[e], preferred_element_type=jnp.float32)
        lg = lg + b2_ref[e][None, :]
        el_ref[:, e, :] = lg
        acc = lg if acc is None else acc + lg
    inv_e = 1.0 / e_count
    mixed_ref[...] = acc * inv_e
    probs_ref[...] = jnp.full(probs_ref.shape, inv_e, dtype=jnp.float32)


def kernel(x, W1, b1, W2, b2):
    n, d = x.shape
    e, _, h = W1.shape
    c = W2.shape[2]
    tn = _TN

    w1b = W1.astype(jnp.bfloat16)
    w2b = W2.astype(jnp.bfloat16)

    mixed, expert_logits, probs = pl.pallas_call(
        _moe_head_kernel,
        grid=(n // tn,),
        in_specs=[
            pl.BlockSpec((tn, d), lambda i: (i, 0)),
            pl.BlockSpec((e, d, h), lambda i: (0, 0, 0)),
            pl.BlockSpec((e, h), lambda i: (0, 0)),
            pl.BlockSpec((e, h, c), lambda i: (0, 0, 0)),
            pl.BlockSpec((e, c), lambda i: (0, 0)),
        ],
        out_specs=[
            pl.BlockSpec((tn, c), lambda i: (i, 0)),
            pl.BlockSpec((tn, e, c), lambda i: (i, 0, 0)),
            pl.BlockSpec((tn, e), lambda i: (i, 0)),
        ],
        out_shape=[
            jax.ShapeDtypeStruct((n, c), jnp.float32),
            jax.ShapeDtypeStruct((n, e, c), jnp.float32),
            jax.ShapeDtypeStruct((n, e), jnp.float32),
        ],
    )(x, w1b, b1, w2b, b2)

    return (mixed, probs, expert_logits, probs)


# speedup vs baseline: 1.5854x; 1.5854x over previous
import jax
import jax.numpy as jnp
from jax.experimental import pallas as pl

_TN = 512


def _moe_head_kernel(x_ref, w1f_ref, b1f_ref, b_ref, b2t_ref, m_ref,
                     mixed_ref, el_ref, probs_ref):
    x = x_ref[...].astype(jnp.bfloat16)
    h = jnp.dot(x, w1f_ref[...], preferred_element_type=jnp.float32)
    h = jnp.maximum(h + b1f_ref[...], 0.0).astype(jnp.bfloat16)
    s = jnp.dot(h, b_ref[...], preferred_element_type=jnp.float32)
    s = s + b2t_ref[...]
    el_ref[...] = s
    mixed_ref[...] = jnp.dot(s.astype(jnp.bfloat16), m_ref[...],
                             preferred_element_type=jnp.float32)
    probs_ref[...] = jnp.full(probs_ref.shape, 1.0 / probs_ref.shape[1],
                              dtype=jnp.float32)


def kernel(x, W1, b1, W2, b2):
    n, d = x.shape
    e, _, h = W1.shape
    c = W2.shape[2]
    tn = _TN
    eh, ec = e * h, e * c

    w1f = jnp.transpose(W1, (1, 0, 2)).reshape(d, eh).astype(jnp.bfloat16)
    b1f = b1.reshape(1, eh)
    bd = jnp.zeros((e, h, e, c), W2.dtype)
    bd = bd.at[jnp.arange(e), :, jnp.arange(e), :].set(W2)
    bd = bd.reshape(eh, ec).astype(jnp.bfloat16)
    b2t = b2.reshape(1, ec)
    m = jnp.tile(jnp.eye(c, dtype=jnp.bfloat16), (e, 1)) * (1.0 / e)

    mixed, el, probs = pl.pallas_call(
        _moe_head_kernel,
        grid=(n // tn,),
        in_specs=[
            pl.BlockSpec((tn, d), lambda i: (i, 0)),
            pl.BlockSpec((d, eh), lambda i: (0, 0)),
            pl.BlockSpec((1, eh), lambda i: (0, 0)),
            pl.BlockSpec((eh, ec), lambda i: (0, 0)),
            pl.BlockSpec((1, ec), lambda i: (0, 0)),
            pl.BlockSpec((ec, c), lambda i: (0, 0)),
        ],
        out_specs=[
            pl.BlockSpec((tn, c), lambda i: (i, 0)),
            pl.BlockSpec((tn, ec), lambda i: (i, 0)),
            pl.BlockSpec((tn, e), lambda i: (i, 0)),
        ],
        out_shape=[
            jax.ShapeDtypeStruct((n, c), jnp.float32),
            jax.ShapeDtypeStruct((n, ec), jnp.float32),
            jax.ShapeDtypeStruct((n, e), jnp.float32),
        ],
    )(x, w1f, b1f, bd, b2t, m)

    expert_logits = el.reshape(n, e, c)
    return (mixed, probs, expert_logits, probs)


# speedup vs baseline: 1.5880x; 1.0016x over previous
import jax
import jax.numpy as jnp
from jax.experimental import pallas as pl
from jax.experimental.pallas import tpu as pltpu

_TN = 512


def _moe_head_kernel(x_ref, w1f_ref, b1f_ref, b_ref, b2t_ref, m_ref,
                     mixed_ref, el_ref, probs_ref):
    x = x_ref[...].astype(jnp.bfloat16)
    h = jnp.dot(x, w1f_ref[...], preferred_element_type=jnp.float32)
    h = jnp.maximum(h + b1f_ref[...], 0.0).astype(jnp.bfloat16)
    s = jnp.dot(h, b_ref[...], preferred_element_type=jnp.float32)
    s = s + b2t_ref[...]
    el_ref[...] = s
    mixed_ref[...] = jnp.dot(s.astype(jnp.bfloat16), m_ref[...],
                             preferred_element_type=jnp.float32)
    probs_ref[...] = jnp.full(probs_ref.shape, 1.0 / probs_ref.shape[1],
                              dtype=jnp.float32)


def kernel(x, W1, b1, W2, b2):
    n, d = x.shape
    e, _, h = W1.shape
    c = W2.shape[2]
    tn = _TN
    eh, ec = e * h, e * c

    w1f = jnp.transpose(W1, (1, 0, 2)).reshape(d, eh).astype(jnp.bfloat16)
    b1f = b1.reshape(1, eh)
    bd = jnp.zeros((e, h, e, c), W2.dtype)
    bd = bd.at[jnp.arange(e), :, jnp.arange(e), :].set(W2)
    bd = bd.reshape(eh, ec).astype(jnp.bfloat16)
    b2t = b2.reshape(1, ec)
    m = jnp.tile(jnp.eye(c, dtype=jnp.bfloat16), (e, 1)) * (1.0 / e)

    mixed, el, probs = pl.pallas_call(
        _moe_head_kernel,
        grid=(n // tn,),
        in_specs=[
            pl.BlockSpec((tn, d), lambda i: (i, 0)),
            pl.BlockSpec((d, eh), lambda i: (0, 0)),
            pl.BlockSpec((1, eh), lambda i: (0, 0)),
            pl.BlockSpec((eh, ec), lambda i: (0, 0)),
            pl.BlockSpec((1, ec), lambda i: (0, 0)),
            pl.BlockSpec((ec, c), lambda i: (0, 0)),
        ],
        out_specs=[
            pl.BlockSpec((tn, c), lambda i: (i, 0)),
            pl.BlockSpec((tn, ec), lambda i: (i, 0)),
            pl.BlockSpec((tn, e), lambda i: (i, 0)),
        ],
        out_shape=[
            jax.ShapeDtypeStruct((n, c), jnp.float32),
            jax.ShapeDtypeStruct((n, ec), jnp.float32),
            jax.ShapeDtypeStruct((n, e), jnp.float32),
        ],
        compiler_params=pltpu.CompilerParams(
            dimension_semantics=("parallel",)),
    )(x, w1f, b1f, bd, b2t, m)

    expert_logits = el.reshape(n, e, c)
    return (mixed, probs, expert_logits, probs)
